# baseline (speedup 1.0000x reference)
import functools

import jax
import jax.numpy as jnp
from jax import lax
from jax.experimental import pallas as pl
from jax.experimental.pallas import tpu as pltpu
from jax.experimental.pallas import tpu_sc as plsc

_B, _S, _D, _P = 8, 2048, 128, 8
_LANES = 16
_NV = _D // _LANES
_WIN = 512
_NC = 4
_CH = _WIN // _NC


def _sc_patch_pool(batch, lens_flat):
    mesh = plsc.VectorSubcoreMesh(core_axis_name="c", subcore_axis_name="s")

    @functools.partial(
        pl.kernel,
        out_type=jax.ShapeDtypeStruct((_B, _P, _D), jnp.float32),
        mesh=mesh,
        compiler_params=pltpu.CompilerParams(
            use_tc_tiling_on_sc=False, needs_layout_passes=False
        ),
        scratch_types=[
            pltpu.VMEM((_LANES,), jnp.int32),
            pltpu.VMEM((_WIN, _D), jnp.float32),
            pltpu.VMEM((_P, _D), jnp.float32),
            pltpu.VMEM((4, 2, _D), jnp.float32),
            pltpu.VMEM((2, _D), jnp.float32),
            pltpu.VMEM_SHARED((16, _P, _D), jnp.float32),
            pltpu.SemaphoreType.DMA,
            pltpu.SemaphoreType.DMA,
            pltpu.SemaphoreType.DMA,
            pltpu.SemaphoreType.DMA,
            pltpu.SemaphoreType.DMA,
            pltpu.SemaphoreType.DMA,
        ],
    )
    def k(batch_hbm, lens_hbm, out_hbm, lens_v, buf_v, pacc_v, fin4_v,
          fin_v, shared_v, lsem, sem0, sem1, sem2, sem3, osem):
        sems = (sem0, sem1, sem2, sem3)
        cid = lax.axis_index("c")
        sid = lax.axis_index("s")
        b = cid * 4 + sid // 4
        bm = b % 4
        base = (sid % 4) * _WIN
        p0f = 2 * (sid % 4)

        start = jnp.minimum(8 * b, 64 - _LANES)
        pltpu.async_copy(lens_hbm.at[pl.ds(start, _LANES)], lens_v, lsem)
        pltpu.async_copy(
            batch_hbm.at[b, pl.ds(base, _CH)], buf_v.at[pl.ds(0, _CH)],
            sems[0],
        )

        with jax.named_scope("ph_lens"):
            pltpu.make_async_copy(
                lens_hbm.at[pl.ds(start, _LANES)], lens_v, lsem
            ).wait()
        iota = lax.iota(jnp.int32, _LANES)
        zero_v = jnp.zeros((_LANES,), jnp.int32)
        zf = jnp.zeros((_LANES,), jnp.float32)
        raw = lens_v[...]
        cums = plsc.cumsum(raw)
        ofs = 8 * b - start
        base0 = jnp.where(
            ofs > 0, jnp.sum(jnp.where(iota == 7, cums, zero_v)), 0
        )
        to_v = cums - base0
        frm_v = to_v - raw
        toc_v = jnp.minimum(to_v, _S)
        frmc_v = jnp.minimum(frm_v, _S)
        n_v = toc_v - frmc_v
        lo_v = jnp.clip(frmc_v - base, 0, _WIN)
        hi_v = jnp.clip(toc_v - base, 0, _WIN)

        def accumulate(p, lo, hi, accs):
            def body(j, accs):
                return tuple(
                    a + buf_v[j, pl.ds(d0 * _LANES, _LANES)]
                    for d0, a in enumerate(accs)
                )

            accs = plsc.parallel_loop(lo, hi, unroll=8, carry=accs)(body)

            for d0, a in enumerate(accs):
                pacc_v[p, pl.ds(d0 * _LANES, _LANES)] = a

        for c in range(_NC):
            with jax.named_scope(f"ph_wait{c}"):
                pltpu.make_async_copy(
                    batch_hbm.at[b, pl.ds(base + c * _CH, _CH)],
                    buf_v.at[pl.ds(c * _CH, _CH)],
                    sems[c],
                ).wait()
            if c + 1 < _NC:
                pltpu.async_copy(
                    batch_hbm.at[b, pl.ds(base + (c + 1) * _CH, _CH)],
                    buf_v.at[pl.ds((c + 1) * _CH, _CH)],
                    sems[c + 1],
                )

            with jax.named_scope(f"ph_acc{c}"):
                @pl.loop(0, _P)
                def _(p, c=c):
                    sel = iota == ofs + p
                    lo = jnp.sum(jnp.where(sel, lo_v, zero_v))
                    hi = jnp.sum(jnp.where(sel, hi_v, zero_v))
                    if c == 0:
                        accs = (zf,) * _NV
                    else:
                        accs = tuple(
                            pacc_v[p, pl.ds(d0 * _LANES, _LANES)]
                            for d0 in range(_NV)
                        )
                    accumulate(p, jnp.clip(lo, c * _CH, (c + 1) * _CH),
                               jnp.clip(hi, c * _CH, (c + 1) * _CH), accs)

        with jax.named_scope("ph_pub"):
            pltpu.sync_copy(pacc_v, shared_v.at[sid])
        with jax.named_scope("ph_barrier"):
            plsc.subcore_barrier()

        with jax.named_scope("ph_fin"):
            pltpu.sync_copy(
                shared_v.at[pl.ds(4 * bm, 4), pl.ds(p0f, 2)], fin4_v
            )
        for i in range(2):
            p = p0f + i
            n = jnp.sum(jnp.where(iota == ofs + p, n_v, zero_v))
            denom = jnp.maximum(n, 1).astype(jnp.float32)
            empty = n == 0
            neg1 = jnp.full((_LANES,), -1.0, jnp.float32)
            for d0 in range(_NV):
                sl = pl.ds(d0 * _LANES, _LANES)
                tot = (fin4_v[0, i, sl] + fin4_v[1, i, sl]
                       + fin4_v[2, i, sl] + fin4_v[3, i, sl])
                fin_v[i, sl] = jnp.where(empty, neg1, tot / denom)
            pltpu.async_copy(fin_v.at[i], out_hbm.at[b, p], osem)

        for i in range(2):
            pltpu.make_async_copy(
                fin_v.at[i], out_hbm.at[b, p0f + i], osem
            ).wait()

    return k(batch, lens_flat)


def kernel(batch, patch_lengths):
    return _sc_patch_pool(batch, jnp.reshape(patch_lengths, (_B * _P,)))


# speedup vs baseline: 1.0575x; 1.0575x over previous
import functools

import jax
import jax.numpy as jnp
from jax import lax
from jax.experimental import pallas as pl
from jax.experimental.pallas import tpu as pltpu
from jax.experimental.pallas import tpu_sc as plsc

_B, _S, _D, _P = 8, 2048, 128, 8
_LANES = 16
_NV = _D // _LANES
_WIN = 512
_HALF = 256


def _sc_patch_pool(batch, lens_flat):
    mesh = plsc.VectorSubcoreMesh(core_axis_name="c", subcore_axis_name="s")

    @functools.partial(
        pl.kernel,
        out_type=jax.ShapeDtypeStruct((_B, _P, _D), jnp.float32),
        mesh=mesh,
        compiler_params=pltpu.CompilerParams(
            use_tc_tiling_on_sc=False, needs_layout_passes=False
        ),
        scratch_types=[
            pltpu.VMEM((_LANES,), jnp.int32),
            pltpu.VMEM((_WIN, _D), jnp.float32),
            pltpu.VMEM((_P, _D), jnp.float32),
            pltpu.VMEM((4, 2, _D), jnp.float32),
            pltpu.VMEM((2, _D), jnp.float32),
            pltpu.VMEM_SHARED((16, _P, _D), jnp.float32),
            pltpu.SemaphoreType.DMA,
            pltpu.SemaphoreType.DMA,
            pltpu.SemaphoreType.DMA,
            pltpu.SemaphoreType.DMA,
        ],
    )
    def k(batch_hbm, lens_hbm, out_hbm, lens_v, buf_v, pacc_v, fin4_v,
          fin_v, shared_v, lsem, sem0, sem1, osem):
        cid = lax.axis_index("c")
        sid = lax.axis_index("s")
        b = cid * 4 + sid // 4
        bm = b % 4
        base = (sid % 4) * _WIN
        p0f = 2 * (sid % 4)

        start = jnp.minimum(8 * b, 64 - _LANES)
        pltpu.async_copy(lens_hbm.at[pl.ds(start, _LANES)], lens_v, lsem)
        pltpu.async_copy(
            batch_hbm.at[b, pl.ds(base, _HALF)], buf_v.at[pl.ds(0, _HALF)],
            sem0,
        )

        pltpu.make_async_copy(
            lens_hbm.at[pl.ds(start, _LANES)], lens_v, lsem
        ).wait()
        iota = lax.iota(jnp.int32, _LANES)
        zero_v = jnp.zeros((_LANES,), jnp.int32)
        zf = jnp.zeros((_LANES,), jnp.float32)
        raw = lens_v[...]
        cums = plsc.cumsum(raw)
        ofs = 8 * b - start
        base0 = jnp.where(
            ofs > 0, jnp.sum(jnp.where(iota == 7, cums, zero_v)), 0
        )
        to_v = cums - base0
        frm_v = to_v - raw
        toc_v = jnp.minimum(to_v, _S)
        frmc_v = jnp.minimum(frm_v, _S)
        n_v = toc_v - frmc_v
        lo_v = jnp.clip(frmc_v - base, 0, _WIN)
        hi_v = jnp.clip(toc_v - base, 0, _WIN)

        def accumulate(p, lo, hi, accs):
            def body(j, accs):
                return tuple(
                    a + buf_v[j, pl.ds(d0 * _LANES, _LANES)]
                    for d0, a in enumerate(accs)
                )

            accs = plsc.parallel_loop(lo, hi, unroll=8, carry=accs)(body)

            for d0, a in enumerate(accs):
                pacc_v[p, pl.ds(d0 * _LANES, _LANES)] = a

        pltpu.make_async_copy(
            batch_hbm.at[b, pl.ds(base, _HALF)], buf_v.at[pl.ds(0, _HALF)],
            sem0,
        ).wait()
        pltpu.async_copy(
            batch_hbm.at[b, pl.ds(base + _HALF, _HALF)],
            buf_v.at[pl.ds(_HALF, _HALF)],
            sem1,
        )

        @pl.loop(0, _P)
        def _(p):
            sel = iota == ofs + p
            lo = jnp.sum(jnp.where(sel, lo_v, zero_v))
            hi = jnp.sum(jnp.where(sel, hi_v, zero_v))
            accumulate(p, jnp.minimum(lo, _HALF), jnp.minimum(hi, _HALF),
                       (zf,) * _NV)

        pltpu.make_async_copy(
            batch_hbm.at[b, pl.ds(base + _HALF, _HALF)],
            buf_v.at[pl.ds(_HALF, _HALF)],
            sem1,
        ).wait()

        @pl.loop(0, _P)
        def _(p):
            sel = iota == ofs + p
            lo = jnp.sum(jnp.where(sel, lo_v, zero_v))
            hi = jnp.sum(jnp.where(sel, hi_v, zero_v))
            accs = tuple(
                pacc_v[p, pl.ds(d0 * _LANES, _LANES)] for d0 in range(_NV)
            )
            accumulate(p, jnp.maximum(lo, _HALF), jnp.maximum(hi, _HALF),
                       accs)

        pltpu.sync_copy(pacc_v, shared_v.at[sid])
        plsc.subcore_barrier()

        pltpu.sync_copy(
            shared_v.at[pl.ds(4 * bm, 4), pl.ds(p0f, 2)], fin4_v
        )
        for i in range(2):
            p = p0f + i
            n = jnp.sum(jnp.where(iota == ofs + p, n_v, zero_v))
            denom = jnp.maximum(n, 1).astype(jnp.float32)
            empty = n == 0
            neg1 = jnp.full((_LANES,), -1.0, jnp.float32)
            for d0 in range(_NV):
                sl = pl.ds(d0 * _LANES, _LANES)
                tot = (fin4_v[0, i, sl] + fin4_v[1, i, sl]
                       + fin4_v[2, i, sl] + fin4_v[3, i, sl])
                fin_v[i, sl] = jnp.where(empty, neg1, tot / denom)
            pltpu.async_copy(fin_v.at[i], out_hbm.at[b, p], osem)

        for i in range(2):
            pltpu.make_async_copy(
                fin_v.at[i], out_hbm.at[b, p0f + i], osem
            ).wait()

    return k(batch, lens_flat)


def kernel(batch, patch_lengths):
    return _sc_patch_pool(batch, jnp.reshape(patch_lengths, (_B * _P,)))


# speedup vs baseline: 1.0608x; 1.0032x over previous
import functools

import jax
import jax.numpy as jnp
from jax import lax
from jax.experimental import pallas as pl
from jax.experimental.pallas import tpu as pltpu
from jax.experimental.pallas import tpu_sc as plsc

_B, _S, _D, _P = 8, 2048, 128, 8
_LANES = 16
_NV = _D // _LANES
_WIN = 512
_HALF = 256


def _sc_patch_pool(batch, lens_flat):
    mesh = plsc.VectorSubcoreMesh(core_axis_name="c", subcore_axis_name="s")

    @functools.partial(
        pl.kernel,
        out_type=jax.ShapeDtypeStruct((_B, _P, _D), jnp.float32),
        mesh=mesh,
        compiler_params=pltpu.CompilerParams(
            use_tc_tiling_on_sc=False, needs_layout_passes=False,
            disable_bounds_checks=True, disable_semaphore_checks=True,
        ),
        scratch_types=[
            pltpu.VMEM((_LANES,), jnp.int32),
            pltpu.VMEM((_WIN, _D), jnp.float32),
            pltpu.VMEM((_P, _D), jnp.float32),
            pltpu.VMEM((4, 2, _D), jnp.float32),
            pltpu.VMEM((2, _D), jnp.float32),
            pltpu.VMEM_SHARED((16, _P, _D), jnp.float32),
            pltpu.SemaphoreType.DMA,
            pltpu.SemaphoreType.DMA,
            pltpu.SemaphoreType.DMA,
            pltpu.SemaphoreType.DMA,
        ],
    )
    def k(batch_hbm, lens_hbm, out_hbm, lens_v, buf_v, pacc_v, fin4_v,
          fin_v, shared_v, lsem, sem0, sem1, osem):
        cid = lax.axis_index("c")
        sid = lax.axis_index("s")
        b = cid * 4 + sid // 4
        bm = b % 4
        base = (sid % 4) * _WIN
        p0f = 2 * (sid % 4)

        start = jnp.minimum(8 * b, 64 - _LANES)
        pltpu.async_copy(lens_hbm.at[pl.ds(start, _LANES)], lens_v, lsem)
        pltpu.async_copy(
            batch_hbm.at[b, pl.ds(base, _HALF)], buf_v.at[pl.ds(0, _HALF)],
            sem0,
        )

        pltpu.make_async_copy(
            lens_hbm.at[pl.ds(start, _LANES)], lens_v, lsem
        ).wait()
        iota = lax.iota(jnp.int32, _LANES)
        zero_v = jnp.zeros((_LANES,), jnp.int32)
        zf = jnp.zeros((_LANES,), jnp.float32)
        raw = lens_v[...]
        cums = plsc.cumsum(raw)
        ofs = 8 * b - start
        base0 = jnp.where(
            ofs > 0, jnp.sum(jnp.where(iota == 7, cums, zero_v)), 0
        )
        to_v = cums - base0
        frm_v = to_v - raw
        toc_v = jnp.minimum(to_v, _S)
        frmc_v = jnp.minimum(frm_v, _S)
        n_v = toc_v - frmc_v
        lo_v = jnp.clip(frmc_v - base, 0, _WIN)
        hi_v = jnp.clip(toc_v - base, 0, _WIN)

        def accumulate(p, lo, hi, accs):
            def body(j, accs):
                return tuple(
                    a + buf_v[j, pl.ds(d0 * _LANES, _LANES)]
                    for d0, a in enumerate(accs)
                )

            accs = plsc.parallel_loop(lo, hi, unroll=8, carry=accs)(body)

            for d0, a in enumerate(accs):
                pacc_v[p, pl.ds(d0 * _LANES, _LANES)] = a

        pltpu.make_async_copy(
            batch_hbm.at[b, pl.ds(base, _HALF)], buf_v.at[pl.ds(0, _HALF)],
            sem0,
        ).wait()
        pltpu.async_copy(
            batch_hbm.at[b, pl.ds(base + _HALF, _HALF)],
            buf_v.at[pl.ds(_HALF, _HALF)],
            sem1,
        )

        @pl.loop(0, _P)
        def _(p):
            sel = iota == ofs + p
            lo = jnp.sum(jnp.where(sel, lo_v, zero_v))
            hi = jnp.sum(jnp.where(sel, hi_v, zero_v))
            accumulate(p, jnp.minimum(lo, _HALF), jnp.minimum(hi, _HALF),
                       (zf,) * _NV)

        pltpu.make_async_copy(
            batch_hbm.at[b, pl.ds(base + _HALF, _HALF)],
            buf_v.at[pl.ds(_HALF, _HALF)],
            sem1,
        ).wait()

        @pl.loop(0, _P)
        def _(p):
            sel = iota == ofs + p
            lo = jnp.sum(jnp.where(sel, lo_v, zero_v))
            hi = jnp.sum(jnp.where(sel, hi_v, zero_v))
            accs = tuple(
                pacc_v[p, pl.ds(d0 * _LANES, _LANES)] for d0 in range(_NV)
            )
            accumulate(p, jnp.maximum(lo, _HALF), jnp.maximum(hi, _HALF),
                       accs)

        pltpu.sync_copy(pacc_v, shared_v.at[sid])
        plsc.subcore_barrier()

        pltpu.sync_copy(
            shared_v.at[pl.ds(4 * bm, 4), pl.ds(p0f, 2)], fin4_v
        )
        for i in range(2):
            p = p0f + i
            n = jnp.sum(jnp.where(iota == ofs + p, n_v, zero_v))
            denom = jnp.maximum(n, 1).astype(jnp.float32)
            empty = n == 0
            neg1 = jnp.full((_LANES,), -1.0, jnp.float32)
            for d0 in range(_NV):
                sl = pl.ds(d0 * _LANES, _LANES)
                tot = (fin4_v[0, i, sl] + fin4_v[1, i, sl]
                       + fin4_v[2, i, sl] + fin4_v[3, i, sl])
                fin_v[i, sl] = jnp.where(empty, neg1, tot / denom)
            pltpu.async_copy(fin_v.at[i], out_hbm.at[b, p], osem)

        for i in range(2):
            pltpu.make_async_copy(
                fin_v.at[i], out_hbm.at[b, p0f + i], osem
            ).wait()

    return k(batch, lens_flat)


def kernel(batch, patch_lengths):
    return _sc_patch_pool(batch, jnp.reshape(patch_lengths, (_B * _P,)))
